# baseline (speedup 1.0000x reference)
import functools

import jax
import jax.numpy as jnp
from jax import lax
from jax.experimental import pallas as pl
from jax.experimental.pallas import tpu as pltpu
from jax.experimental.pallas import tpu_sc as plsc

B_STATIC = 4
B_TC = 2
B_SC = B_STATIC - B_TC
CHUNK = 64
BT = 256


def _bcast_body(w_ref, o_ref):
    o_ref[...] = jnp.broadcast_to(w_ref[...][None], o_ref.shape)


def kernel(B, T, pe_weight):
    max_len, d_model = pe_weight.shape
    info = plsc.get_sparse_core_info()
    nc, ns = info.num_cores, info.num_subcores
    nw = nc * ns
    rows = max_len // nw
    nchunks = rows // CHUNK

    mesh = plsc.VectorSubcoreMesh(core_axis_name="c", subcore_axis_name="s")

    @functools.partial(
        pl.kernel,
        mesh=mesh,
        out_type=jax.ShapeDtypeStruct((B_SC, max_len, d_model), pe_weight.dtype),
        scratch_types=[
            pltpu.VMEM((CHUNK, d_model), pe_weight.dtype),
            pltpu.SemaphoreType.DMA,
        ],
    )
    def sc_copy(table_hbm, out_hbm, buf, sem):
        wid = lax.axis_index("s") * nc + lax.axis_index("c")
        base = wid * rows

        def body(i, carry):
            start = base + i * CHUNK
            pltpu.sync_copy(table_hbm.at[pl.ds(start, CHUNK)], buf)
            copies = [
                pltpu.async_copy(buf, out_hbm.at[b, pl.ds(start, CHUNK)], sem)
                for b in range(B_SC)
            ]
            for c in copies:
                c.wait()
            return carry

        lax.fori_loop(0, nchunks, body, 0)

    out_sc = sc_copy(pe_weight)

    out_tc = pl.pallas_call(
        _bcast_body,
        grid=(max_len // BT,),
        in_specs=[pl.BlockSpec((BT, d_model), lambda i: (i, 0))],
        out_specs=pl.BlockSpec((B_TC, BT, d_model), lambda i: (0, i, 0)),
        out_shape=jax.ShapeDtypeStruct((B_TC, max_len, d_model), pe_weight.dtype),
    )(pe_weight)

    return jnp.concatenate([out_tc, out_sc], axis=0)


# speedup vs baseline: 2.2889x; 2.2889x over previous
import functools

import jax
import jax.numpy as jnp
from jax import lax
from jax.experimental import pallas as pl
from jax.experimental.pallas import tpu as pltpu
from jax.experimental.pallas import tpu_sc as plsc

B_STATIC = 4
CHUNK = 64


def kernel(B, T, pe_weight):
    max_len, d_model = pe_weight.shape
    info = plsc.get_sparse_core_info()
    nc, ns = info.num_cores, info.num_subcores
    nw = nc * ns
    rows = max_len // nw
    nchunks = rows // CHUNK

    mesh = plsc.VectorSubcoreMesh(core_axis_name="c", subcore_axis_name="s")

    @functools.partial(
        pl.kernel,
        mesh=mesh,
        out_type=jax.ShapeDtypeStruct((B_STATIC, max_len, d_model), pe_weight.dtype),
        scratch_types=[
            pltpu.VMEM((CHUNK, d_model), pe_weight.dtype),
            pltpu.SemaphoreType.DMA,
        ],
    )
    def sc_copy(table_hbm, out_hbm, buf, sem):
        wid = lax.axis_index("s") * nc + lax.axis_index("c")
        base = wid * rows

        for i in range(nchunks):
            start = base + i * CHUNK
            pltpu.sync_copy(table_hbm.at[pl.ds(start, CHUNK)], buf)
            copies = [
                pltpu.async_copy(buf, out_hbm.at[b, pl.ds(start, CHUNK)], sem)
                for b in range(B_STATIC)
            ]
            for c in copies:
                c.wait()

    return sc_copy(pe_weight)
